# baseline (speedup 1.0000x reference)
import functools

import jax
import jax.numpy as jnp
from jax import lax
from jax.experimental import pallas as pl
from jax.experimental.pallas import tpu as pltpu
from jax.experimental.pallas import tpu_sc as plsc

_W = 8064
_NSTAG = 128


def _build_sc_call(n, num_cores, num_subcores):
    nw = num_cores * num_subcores
    n_groups = n // 8
    gpw = n_groups // nw
    mesh = plsc.VectorSubcoreMesh(core_axis_name="c", subcore_axis_name="s")

    @functools.partial(
        pl.kernel,
        mesh=mesh,
        out_type=jax.ShapeDtypeStruct((n, n), jnp.float32),
        scratch_types=[
            pltpu.VMEM((8, _W), jnp.float32),
            pltpu.SemaphoreType.DMA,
            pltpu.SemaphoreType.DMA,
        ],
    )
    def run(mega_hbm, out_hbm, fs_v, load_sem, row_sem):
        wid = lax.axis_index("s") * num_cores + lax.axis_index("c")
        r16 = wid % 16
        half = wid // 16
        pltpu.async_copy(mega_hbm.at[r16], fs_v, load_sem).wait()
        descs = []
        for k in range(gpw):
            m_idx = gpw * half + k
            row0 = 8 * r16 + 128 * m_idx
            qp = 128 * (31 - m_idx)
            descs.append(
                pltpu.async_copy(
                    fs_v.at[:, pl.ds(qp, n)],
                    out_hbm.at[pl.ds(row0, 8)],
                    row_sem,
                )
            )
        for d in descs:
            d.wait()

    return run


def kernel(query_len, key_len, bias_embedding_table):
    n = bias_embedding_table.shape[0]
    flat = bias_embedding_table[:, 0]
    f_ext = jnp.concatenate([flat[::-1], jnp.full((n,), flat[0], flat.dtype)])
    win = jnp.stack([f_ext[p:p + _W] for p in range(_NSTAG)])
    mega = win[::-1].reshape(16, 8, _W)
    info = plsc.get_sparse_core_info()
    run = _build_sc_call(n, info.num_cores, info.num_subcores)
    return run(mega.astype(jnp.float32))


# speedup vs baseline: 2.7242x; 2.7242x over previous
import functools

import jax
import jax.numpy as jnp
from jax import lax
from jax.experimental import pallas as pl
from jax.experimental.pallas import tpu as pltpu
from jax.experimental.pallas import tpu_sc as plsc

_W = 8064
_NSTAG = 128


def _build_sc_call(n, num_cores, num_subcores):
    nw = num_cores * num_subcores
    n_groups = n // 8
    gpw = n_groups // nw
    mesh = plsc.VectorSubcoreMesh(core_axis_name="c", subcore_axis_name="s")

    @functools.partial(
        pl.kernel,
        mesh=mesh,
        out_type=jax.ShapeDtypeStruct((n, n), jnp.float32),
        scratch_types=[
            pltpu.VMEM((8, _W), jnp.float32),
            pltpu.SemaphoreType.DMA,
            pltpu.SemaphoreType.DMA,
        ],
    )
    def run(mega_hbm, out_hbm, fs_v, load_sem, row_sem):
        wid = lax.axis_index("s") * num_cores + lax.axis_index("c")
        r16 = wid % 16
        half = wid // 16
        pltpu.async_copy(mega_hbm.at[r16], fs_v, load_sem).wait()
        descs = []
        for k in range(gpw):
            m_idx = gpw * half + k
            row0 = 8 * r16 + 128 * m_idx
            qp = 128 * (31 - m_idx)
            descs.append(
                pltpu.async_copy(
                    fs_v.at[:, pl.ds(qp, n)],
                    out_hbm.at[pl.ds(row0, 8)],
                    row_sem,
                )
            )
        for d in descs:
            d.wait()

    return run


def kernel(query_len, key_len, bias_embedding_table):
    n = bias_embedding_table.shape[0]
    flat = bias_embedding_table[:, 0]
    f_ext = jnp.concatenate([flat[::-1], jnp.full((n,), flat[0], flat.dtype)])
    p2 = jnp.roll(f_ext, -(_NSTAG - 1))
    sheared = jnp.tile(p2, _NSTAG)[: _NSTAG * (2 * n - 1)]
    mega = sheared.reshape(_NSTAG, 2 * n - 1)[:, :_W].reshape(16, 8, _W)
    info = plsc.get_sparse_core_info()
    run = _build_sc_call(n, info.num_cores, info.num_subcores)
    return run(mega.astype(jnp.float32))
